# baseline (speedup 1.0000x reference)
import functools

import jax
import jax.numpy as jnp
from jax.experimental import pallas as pl
from jax.experimental.pallas import tpu as pltpu


def _pick_bm(n: int, target: int) -> int:
    bm = 8
    for cand in range(8, target + 1, 8):
        if n % cand == 0:
            bm = cand
    return bm


def _call_a_kernel(x_ref, adj_ref, w0_ref, b0_ref, w1_ref, m1_ref,
                   adjb_ref, s1_ref, z1_ref, s0_scr):
    i = pl.program_id(0)

    @pl.when(i == 0)
    def _():
        xb = x_ref[...].astype(jnp.bfloat16)
        s0_scr[...] = jnp.dot(
            xb, w0_ref[...], preferred_element_type=jnp.float32
        ).astype(jnp.bfloat16)

    @pl.when(i > 0)
    def _():
        a = adj_ref[...].astype(jnp.bfloat16)
        adjb_ref[...] = a
        out = jnp.dot(a, s0_scr[...], preferred_element_type=jnp.float32)
        h = jnp.maximum(out + b0_ref[...], 0.0)
        hb = h.astype(jnp.bfloat16)
        s1_ref[...] = jnp.dot(
            hb, w1_ref[...], preferred_element_type=jnp.float32
        ).astype(jnp.bfloat16)
        z1_ref[...] = jnp.dot(hb, m1_ref[...],
                              preferred_element_type=jnp.float32)


def _call_b_kernel(adjb_ref, s1_ref, z1_ref, b_ref, w2_ref, ml_ref, bl_ref,
                   o_ref, s2_scr, z2_scr, *, bm):
    p = pl.program_id(0)
    j = pl.program_id(1)
    a = adjb_ref[...]

    @pl.when(p == 0)
    def _():
        out = jnp.dot(a, s1_ref[...], preferred_element_type=jnp.float32)
        h = jnp.maximum(out + b_ref[0], 0.0)
        hb = h.astype(jnp.bfloat16)
        s2_scr[pl.ds(j * bm, bm), :] = jnp.dot(
            hb, w2_ref[...], preferred_element_type=jnp.float32
        ).astype(jnp.bfloat16)
        z2_scr[pl.ds(j * bm, bm), :] = jnp.dot(
            hb, ml_ref[0], preferred_element_type=jnp.float32)

    @pl.when(p == 1)
    def _():
        out = jnp.dot(a, s2_scr[...], preferred_element_type=jnp.float32)
        h = jnp.maximum(out + b_ref[0], 0.0)
        hb = h.astype(jnp.bfloat16)
        z3 = jnp.dot(hb, ml_ref[0], preferred_element_type=jnp.float32)
        z = z1_ref[...] + z2_scr[pl.ds(j * bm, bm), :] + z3 + bl_ref[...]
        m = jnp.max(z, axis=1, keepdims=True)
        zs = z - m
        o_ref[0] = zs - jnp.log(jnp.sum(jnp.exp(zs), axis=1, keepdims=True))


def kernel(x, adj, W0, b0, W1, b1, W2, b2, Wl, bl):
    n, f = x.shape
    nclass = Wl.shape[0]

    w0b = W0.astype(jnp.bfloat16)
    w1b = W1.astype(jnp.bfloat16)
    w2b = W2.astype(jnp.bfloat16)
    wlt = Wl.T
    m1 = wlt[0 * f:1 * f].astype(jnp.bfloat16)
    m23 = jnp.stack([wlt[1 * f:2 * f], wlt[2 * f:3 * f]]).astype(jnp.bfloat16)
    b0r = b0.reshape(1, f)
    b12 = jnp.stack([b1.reshape(1, f), b2.reshape(1, f)])
    blr = bl.reshape(1, nclass)

    bm1 = _pick_bm(n, 400)
    g1 = n // bm1
    adjb, s1, z1 = pl.pallas_call(
        _call_a_kernel,
        grid=(g1 + 1,),
        in_specs=[
            pl.BlockSpec((n, f), lambda i: (0, 0)),
            pl.BlockSpec((bm1, n), lambda i: (jnp.maximum(i - 1, 0), 0)),
            pl.BlockSpec((f, f), lambda i: (0, 0)),
            pl.BlockSpec((1, f), lambda i: (0, 0)),
            pl.BlockSpec((f, f), lambda i: (0, 0)),
            pl.BlockSpec((f, nclass), lambda i: (0, 0)),
        ],
        out_specs=(
            pl.BlockSpec((bm1, n), lambda i: (jnp.maximum(i - 1, 0), 0)),
            pl.BlockSpec((bm1, f), lambda i: (jnp.maximum(i - 1, 0), 0)),
            pl.BlockSpec((bm1, nclass), lambda i: (jnp.maximum(i - 1, 0), 0)),
        ),
        out_shape=(
            jax.ShapeDtypeStruct((n, n), jnp.bfloat16),
            jax.ShapeDtypeStruct((n, f), jnp.bfloat16),
            jax.ShapeDtypeStruct((n, nclass), jnp.float32),
        ),
        scratch_shapes=[pltpu.VMEM((n, f), jnp.bfloat16)],
    )(x, adj, w0b, b0r, w1b, m1)

    bm = _pick_bm(n, 400)
    g = n // bm
    out = pl.pallas_call(
        functools.partial(_call_b_kernel, bm=bm),
        grid=(2, g),
        in_specs=[
            pl.BlockSpec((bm, n), lambda p, j: (j, 0)),
            pl.BlockSpec((n, f), lambda p, j: (0, 0)),
            pl.BlockSpec((bm, nclass), lambda p, j: (j, 0)),
            pl.BlockSpec((1, 1, f), lambda p, j: (p, 0, 0)),
            pl.BlockSpec((f, f), lambda p, j: (0, 0)),
            pl.BlockSpec((1, f, nclass), lambda p, j: (p, 0, 0)),
            pl.BlockSpec((1, nclass), lambda p, j: (0, 0)),
        ],
        out_specs=pl.BlockSpec((1, bm, nclass), lambda p, j: (p, j, 0)),
        out_shape=jax.ShapeDtypeStruct((2, n, nclass), jnp.float32),
        scratch_shapes=[
            pltpu.VMEM((n, f), jnp.bfloat16),
            pltpu.VMEM((n, nclass), jnp.float32),
        ],
    )(adjb, s1, z1, b12, w2b, m23, blr)
    return out[1]


# speedup vs baseline: 1.0700x; 1.0700x over previous
import functools

import jax
import jax.numpy as jnp
from jax.experimental import pallas as pl
from jax.experimental.pallas import tpu as pltpu


def _pick_bm(n: int, target: int) -> int:
    bm = 8
    for cand in range(8, target + 1, 8):
        if n % cand == 0:
            bm = cand
    return bm


def _call_a_kernel(x_ref, adj_ref, w0_ref, b0_ref, w1_ref, m1_ref,
                   adjb_ref, s1_ref, z1_ref, s0_scr):
    i = pl.program_id(0)

    @pl.when(i == 0)
    def _():
        xb = x_ref[...].astype(jnp.bfloat16)
        s0_scr[...] = jnp.dot(
            xb, w0_ref[...], preferred_element_type=jnp.float32
        ).astype(jnp.bfloat16)

    @pl.when(i > 0)
    def _():
        a = adj_ref[...].astype(jnp.bfloat16)
        adjb_ref[...] = a
        out = jnp.dot(a, s0_scr[...], preferred_element_type=jnp.float32)
        h = jnp.maximum(out + b0_ref[...], 0.0)
        hb = h.astype(jnp.bfloat16)
        s1_ref[...] = jnp.dot(
            hb, w1_ref[...], preferred_element_type=jnp.float32
        ).astype(jnp.bfloat16)
        z1_ref[...] = jnp.dot(hb, m1_ref[...],
                              preferred_element_type=jnp.float32)


def _layer2_kernel(adjb_ref, s_ref, b_ref, wn_ref, ml_ref, sn_ref, z_ref):
    a = adjb_ref[...]
    out = jnp.dot(a, s_ref[...], preferred_element_type=jnp.float32)
    h = jnp.maximum(out + b_ref[...], 0.0)
    hb = h.astype(jnp.bfloat16)
    sn_ref[...] = jnp.dot(
        hb, wn_ref[...], preferred_element_type=jnp.float32
    ).astype(jnp.bfloat16)
    z_ref[...] = jnp.dot(hb, ml_ref[...], preferred_element_type=jnp.float32)


def _layer3_final_kernel(adjb_ref, s_ref, b_ref, ml_ref, z1_ref, z2_ref,
                         bl_ref, o_ref):
    a = adjb_ref[...]
    out = jnp.dot(a, s_ref[...], preferred_element_type=jnp.float32)
    h = jnp.maximum(out + b_ref[...], 0.0)
    hb = h.astype(jnp.bfloat16)
    z3 = jnp.dot(hb, ml_ref[...], preferred_element_type=jnp.float32)
    z = z1_ref[...] + z2_ref[...] + z3 + bl_ref[...]
    m = jnp.max(z, axis=1, keepdims=True)
    zs = z - m
    o_ref[...] = zs - jnp.log(jnp.sum(jnp.exp(zs), axis=1, keepdims=True))


def kernel(x, adj, W0, b0, W1, b1, W2, b2, Wl, bl):
    n, f = x.shape
    nclass = Wl.shape[0]

    w0b = W0.astype(jnp.bfloat16)
    w1b = W1.astype(jnp.bfloat16)
    w2b = W2.astype(jnp.bfloat16)
    wlt = Wl.T
    m1 = wlt[0 * f:1 * f].astype(jnp.bfloat16)
    m2 = wlt[1 * f:2 * f].astype(jnp.bfloat16)
    m3 = wlt[2 * f:3 * f].astype(jnp.bfloat16)
    b0r = b0.reshape(1, f)
    b1r = b1.reshape(1, f)
    b2r = b2.reshape(1, f)
    blr = bl.reshape(1, nclass)

    bm1 = _pick_bm(n, 400)
    g1 = n // bm1
    adjb, s1, z1 = pl.pallas_call(
        _call_a_kernel,
        grid=(g1 + 1,),
        in_specs=[
            pl.BlockSpec((n, f), lambda i: (0, 0)),
            pl.BlockSpec((bm1, n), lambda i: (jnp.maximum(i - 1, 0), 0)),
            pl.BlockSpec((f, f), lambda i: (0, 0)),
            pl.BlockSpec((1, f), lambda i: (0, 0)),
            pl.BlockSpec((f, f), lambda i: (0, 0)),
            pl.BlockSpec((f, nclass), lambda i: (0, 0)),
        ],
        out_specs=(
            pl.BlockSpec((bm1, n), lambda i: (jnp.maximum(i - 1, 0), 0)),
            pl.BlockSpec((bm1, f), lambda i: (jnp.maximum(i - 1, 0), 0)),
            pl.BlockSpec((bm1, nclass), lambda i: (jnp.maximum(i - 1, 0), 0)),
        ),
        out_shape=(
            jax.ShapeDtypeStruct((n, n), jnp.bfloat16),
            jax.ShapeDtypeStruct((n, f), jnp.bfloat16),
            jax.ShapeDtypeStruct((n, nclass), jnp.float32),
        ),
        scratch_shapes=[pltpu.VMEM((n, f), jnp.bfloat16)],
    )(x, adj, w0b, b0r, w1b, m1)

    bm = _pick_bm(n, 1000)
    g = n // bm
    s2, z2 = pl.pallas_call(
        _layer2_kernel,
        grid=(g,),
        in_specs=[
            pl.BlockSpec((bm, n), lambda j: (j, 0)),
            pl.BlockSpec((n, f), lambda j: (0, 0)),
            pl.BlockSpec((1, f), lambda j: (0, 0)),
            pl.BlockSpec((f, f), lambda j: (0, 0)),
            pl.BlockSpec((f, nclass), lambda j: (0, 0)),
        ],
        out_specs=(
            pl.BlockSpec((bm, f), lambda j: (j, 0)),
            pl.BlockSpec((bm, nclass), lambda j: (j, 0)),
        ),
        out_shape=(
            jax.ShapeDtypeStruct((n, f), jnp.bfloat16),
            jax.ShapeDtypeStruct((n, nclass), jnp.float32),
        ),
    )(adjb, s1, b1r, w2b, m2)

    out = pl.pallas_call(
        _layer3_final_kernel,
        grid=(g,),
        in_specs=[
            pl.BlockSpec((bm, n), lambda j: (j, 0)),
            pl.BlockSpec((n, f), lambda j: (0, 0)),
            pl.BlockSpec((1, f), lambda j: (0, 0)),
            pl.BlockSpec((f, nclass), lambda j: (0, 0)),
            pl.BlockSpec((bm, nclass), lambda j: (j, 0)),
            pl.BlockSpec((bm, nclass), lambda j: (j, 0)),
            pl.BlockSpec((1, nclass), lambda j: (0, 0)),
        ],
        out_specs=pl.BlockSpec((bm, nclass), lambda j: (j, 0)),
        out_shape=jax.ShapeDtypeStruct((n, nclass), jnp.float32),
    )(adjb, s2, b2r, m3, z1, z2, blr)
    return out


# speedup vs baseline: 1.0841x; 1.0132x over previous
import functools

import jax
import jax.numpy as jnp
from jax.experimental import pallas as pl
from jax.experimental.pallas import tpu as pltpu


def _pick_bm(n: int, target: int) -> int:
    bm = 8
    for cand in range(8, target + 1, 8):
        if n % cand == 0:
            bm = cand
    return bm


def _call_a_kernel(x_ref, adj_ref, w0_ref, b0_ref, w1_ref, m1_ref,
                   adjb_ref, s1_ref, z1_ref, s0_scr):
    i = pl.program_id(0)

    @pl.when(i == 0)
    def _():
        xb = x_ref[...].astype(jnp.bfloat16)
        s0_scr[...] = jnp.dot(
            xb, w0_ref[...], preferred_element_type=jnp.float32
        ).astype(jnp.bfloat16)

    @pl.when(i > 0)
    def _():
        a = adj_ref[...].astype(jnp.bfloat16)
        adjb_ref[...] = a
        out = jnp.dot(a, s0_scr[...], preferred_element_type=jnp.float32)
        h = jnp.maximum(out + b0_ref[...], 0.0)
        hb = h.astype(jnp.bfloat16)
        s1_ref[...] = jnp.dot(
            hb, w1_ref[...], preferred_element_type=jnp.float32
        ).astype(jnp.bfloat16)
        z1_ref[...] = jnp.dot(hb, m1_ref[...],
                              preferred_element_type=jnp.float32)


def _call_b_kernel(adjb_ref, s1_ref, z1_ref, b_ref, w2_ref, ml_ref, bl_ref,
                   o_ref, s2_scr, z2_scr, *, bm):
    p = pl.program_id(0)
    j = pl.program_id(1)

    @pl.when(p == 0)
    def _():
        a = adjb_ref[...]
        out = jnp.dot(a, s1_ref[...], preferred_element_type=jnp.float32)
        h = jnp.maximum(out + b_ref[0], 0.0)
        hb = h.astype(jnp.bfloat16)
        s2_scr[pl.ds(j * bm, bm), :] = jnp.dot(
            hb, w2_ref[...], preferred_element_type=jnp.float32
        ).astype(jnp.bfloat16)
        z2_scr[pl.ds(j * bm, bm), :] = jnp.dot(
            hb, ml_ref[0], preferred_element_type=jnp.float32)

    @pl.when(p == 1)
    def _():
        a = adjb_ref[...]
        out = jnp.dot(a, s2_scr[...], preferred_element_type=jnp.float32)
        h = jnp.maximum(out + b_ref[0], 0.0)
        hb = h.astype(jnp.bfloat16)
        z3 = jnp.dot(hb, ml_ref[0], preferred_element_type=jnp.float32)
        z = z1_ref[...] + z2_scr[pl.ds(j * bm, bm), :] + z3 + bl_ref[...]
        m = jnp.max(z, axis=1, keepdims=True)
        zs = z - m
        o_ref[0] = zs - jnp.log(jnp.sum(jnp.exp(zs), axis=1, keepdims=True))


def kernel(x, adj, W0, b0, W1, b1, W2, b2, Wl, bl):
    n, f = x.shape
    nclass = Wl.shape[0]

    w0b = W0.astype(jnp.bfloat16)
    w1b = W1.astype(jnp.bfloat16)
    w2b = W2.astype(jnp.bfloat16)
    wlt = Wl.T
    m1 = wlt[0 * f:1 * f].astype(jnp.bfloat16)
    m2 = wlt[1 * f:2 * f].astype(jnp.bfloat16)
    m3 = wlt[2 * f:3 * f].astype(jnp.bfloat16)
    b0r = b0.reshape(1, f)
    b1r = b1.reshape(1, f)
    b2r = b2.reshape(1, f)
    blr = bl.reshape(1, nclass)

    bm1 = _pick_bm(n, 400)
    g1 = n // bm1
    adjb, s1, z1 = pl.pallas_call(
        _call_a_kernel,
        grid=(g1 + 1,),
        in_specs=[
            pl.BlockSpec((n, f), lambda i: (0, 0)),
            pl.BlockSpec((bm1, n), lambda i: (jnp.maximum(i - 1, 0), 0)),
            pl.BlockSpec((f, f), lambda i: (0, 0)),
            pl.BlockSpec((1, f), lambda i: (0, 0)),
            pl.BlockSpec((f, f), lambda i: (0, 0)),
            pl.BlockSpec((f, nclass), lambda i: (0, 0)),
        ],
        out_specs=(
            pl.BlockSpec((bm1, n), lambda i: (jnp.maximum(i - 1, 0), 0)),
            pl.BlockSpec((bm1, f), lambda i: (jnp.maximum(i - 1, 0), 0)),
            pl.BlockSpec((bm1, nclass), lambda i: (jnp.maximum(i - 1, 0), 0)),
        ),
        out_shape=(
            jax.ShapeDtypeStruct((n, n), jnp.bfloat16),
            jax.ShapeDtypeStruct((n, f), jnp.bfloat16),
            jax.ShapeDtypeStruct((n, nclass), jnp.float32),
        ),
        scratch_shapes=[pltpu.VMEM((n, f), jnp.bfloat16)],
    )(x, adj, w0b, b0r, w1b, m1)

    bm = _pick_bm(n, 1000)
    g = n // bm
    b12 = jnp.stack([b1.reshape(1, f), b2.reshape(1, f)])
    m23 = jnp.stack([m2, m3])
    out = pl.pallas_call(
        functools.partial(_call_b_kernel, bm=bm),
        grid=(2, g),
        in_specs=[
            pl.BlockSpec((bm, n), lambda p, j: (j, 0)),
            pl.BlockSpec((n, f), lambda p, j: (0, 0)),
            pl.BlockSpec((bm, nclass), lambda p, j: (j, 0)),
            pl.BlockSpec((1, 1, f), lambda p, j: (p, 0, 0)),
            pl.BlockSpec((f, f), lambda p, j: (0, 0)),
            pl.BlockSpec((1, f, nclass), lambda p, j: (p, 0, 0)),
            pl.BlockSpec((1, nclass), lambda p, j: (0, 0)),
        ],
        out_specs=pl.BlockSpec((1, bm, nclass), lambda p, j: (p, j, 0)),
        out_shape=jax.ShapeDtypeStruct((2, n, nclass), jnp.float32),
        scratch_shapes=[
            pltpu.VMEM((n, f), jnp.bfloat16),
            pltpu.VMEM((n, nclass), jnp.float32),
        ],
    )(adjb, s1, z1, b12, w2b, m23, blr)
    return out[1]
